# baseline (device time: 44779 ns/iter reference)
import jax
import jax.numpy as jnp
from jax import lax
from jax.experimental import pallas as pl
from jax.experimental.pallas import tpu as pltpu

N_DEV = 8
B = 2
S = 256
R = B * S
D_MODEL = 512
HPB = 4
DH = 64
HB = HPB * DH
NBLK = B * (S // DH)


def kernel(x, Wq, K_ext, V_ext, Wo):
    K_t = jnp.transpose(K_ext, (0, 2, 1, 3)).astype(jnp.bfloat16)
    V_t = jnp.transpose(V_ext, (0, 2, 1, 3)).astype(jnp.bfloat16)
    x_b = x.astype(jnp.bfloat16)
    Wq_b = Wq.astype(jnp.bfloat16)
    Wo_b = Wo.astype(jnp.bfloat16)

    def body(x_ref, wq_ref, k_ref, v_ref, wo_ref, out_ref,
             wq_comm, wo_comm, swq, rwq, swo, rwo):
        my_pos = lax.axis_index("i")

        q = lax.rem(my_pos, 4)
        z4 = my_pos - q
        qx = z4 + (q + 1 - 2 * lax.rem(q, 2))
        qy = z4 + (3 - q)
        qxy = z4 + lax.rem(q + 2, 4)
        partner = {
            1: qx,
            2: qy,
            3: qxy,
            4: lax.rem(my_pos + 4, 8),
            5: lax.rem(qx + 4, 8),
            6: lax.rem(qy + 4, 8),
            7: lax.rem(qxy + 4, 8),
        }

        barrier_sem = pltpu.get_barrier_semaphore()
        for s in range(1, N_DEV):
            pl.semaphore_signal(
                barrier_sem, inc=1,
                device_id=(partner[s],), device_id_type=pl.DeviceIdType.MESH,
            )
        pl.semaphore_wait(barrier_sem, N_DEV - 1)

        x2 = x_ref[...].reshape(R, D_MODEL)

        def compute(slot):
            head0 = (my_pos if slot == 0 else partner[slot]) * HPB
            wq = wq_ref[...] if slot == 0 else wq_comm[slot]
            wo = wo_ref[...] if slot == 0 else wo_comm[slot]
            qp = jnp.dot(x2, wq,
                         preferred_element_type=jnp.float32)
            kb = k_ref[:, pl.ds(head0, HPB)]
            vb = v_ref[:, pl.ds(head0, HPB)]
            ctx_parts = []
            for h in range(HPB):
                qh = (qp[:, h * DH:(h + 1) * DH]
                      .astype(jnp.bfloat16).reshape(NBLK, DH, DH))
                kh = kb[:, h].reshape(NBLK, DH, DH)
                vh = vb[:, h].reshape(NBLK, DH, DH)
                scores = lax.dot_general(
                    qh, kh, (((2,), (2,)), ((0,), (0,))),
                    preferred_element_type=jnp.float32,
                ) * 0.125
                e = jnp.exp(scores)
                w = (e / jnp.sum(e, axis=2, keepdims=True)
                     ).astype(jnp.bfloat16)
                ctx_parts.append(
                    lax.dot_general(
                        w, vh, (((2,), (1,)), ((0,), (0,))),
                        preferred_element_type=jnp.float32,
                    ).reshape(R, DH).astype(jnp.bfloat16))
            ctx = jnp.concatenate(ctx_parts, axis=1)
            contrib = jnp.dot(ctx, wo,
                              preferred_element_type=jnp.float32)
            c3 = contrib.reshape(B, S, D_MODEL)
            if slot == 0:
                out_ref[...] = c3
            else:
                out_ref[...] = out_ref[...] + c3

        def send(slot):
            rq = pltpu.make_async_remote_copy(
                src_ref=wq_ref, dst_ref=wq_comm.at[slot],
                send_sem=swq.at[slot], recv_sem=rwq.at[slot],
                device_id=(partner[slot],),
                device_id_type=pl.DeviceIdType.MESH,
            )
            ro = pltpu.make_async_remote_copy(
                src_ref=wo_ref, dst_ref=wo_comm.at[slot],
                send_sem=swo.at[slot], recv_sem=rwo.at[slot],
                device_id=(partner[slot],),
                device_id_type=pl.DeviceIdType.MESH,
            )
            rq.start()
            ro.start()
            return rq, ro

        def wait_send(pair):
            pair[0].wait_send()
            pair[1].wait_send()

        def wait_recv(pair):
            pair[0].wait_recv()
            pair[1].wait_recv()

        r = {s: send(s) for s in (4, 1, 2)}
        compute(0)
        wait_send(r[1])
        r[3] = send(3)
        wait_recv(r[1])
        compute(1)
        wait_send(r[2])
        r[6] = send(6)
        wait_recv(r[2])
        compute(2)
        wait_send(r[3])
        r[5] = send(5)
        wait_recv(r[3])
        compute(3)
        wait_recv(r[4])
        compute(4)
        wait_send(r[5])
        r[7] = send(7)
        wait_recv(r[6])
        compute(6)
        wait_recv(r[5])
        compute(5)
        wait_recv(r[7])
        compute(7)
        wait_send(r[4])
        wait_send(r[6])
        wait_send(r[7])

    return pl.pallas_call(
        body,
        out_shape=jax.ShapeDtypeStruct((B, S, D_MODEL), jnp.float32),
        in_specs=[pl.BlockSpec(memory_space=pltpu.VMEM)] * 5,
        out_specs=pl.BlockSpec(memory_space=pltpu.VMEM),
        scratch_shapes=[
            pltpu.VMEM((N_DEV, D_MODEL, HB), jnp.bfloat16),
            pltpu.VMEM((N_DEV, HB, D_MODEL), jnp.bfloat16),
            pltpu.SemaphoreType.DMA((N_DEV,)),
            pltpu.SemaphoreType.DMA((N_DEV,)),
            pltpu.SemaphoreType.DMA((N_DEV,)),
            pltpu.SemaphoreType.DMA((N_DEV,)),
        ],
        compiler_params=pltpu.CompilerParams(collective_id=0),
    )(x_b, Wq_b, K_t, V_t, Wo_b)
